# baseline (speedup 1.0000x reference)
import functools

import jax
import jax.numpy as jnp
from jax import lax
from jax.experimental import pallas as pl

_K = 256
_LANE = 128


def _rup(n, m):
    return (n + m - 1) // m * m


def _mm_body(a_ref, b_ref, bias_ref, o_ref):
    o_ref[...] = (
        jnp.dot(a_ref[...], b_ref[...], preferred_element_type=jnp.float32)
        + bias_ref[...]
    )


def _mm(a, b, bias=None):
    m, k = a.shape
    n = b.shape[1]
    if bias is None:
        bias = jnp.zeros((n,), jnp.float32)
    return pl.pallas_call(
        _mm_body,
        out_shape=jax.ShapeDtypeStruct((m, n), jnp.float32),
    )(a, b, bias.reshape(1, n))


def _qsns_body(pooled_ref, ah_ref, wq_ref, bq_ref, ws1_ref, ws2_ref, o_ref):
    q = (
        jnp.dot(pooled_ref[...], wq_ref[...], preferred_element_type=jnp.float32)
        + bq_ref[...]
    )
    qs = jnp.dot(q, ws1_ref[...], preferred_element_type=jnp.float32)
    ns = jnp.dot(ah_ref[...], ws2_ref[...], preferred_element_type=jnp.float32)
    o_ref[:, 0:1] = qs
    o_ref[:, 1:2] = ns


def _qsns(pooled, ah, Wq, bq, ws1, ws2):
    n = pooled.shape[0]
    return pl.pallas_call(
        _qsns_body,
        out_shape=jax.ShapeDtypeStruct((n, 2), jnp.float32),
    )(pooled, ah, Wq, bq.reshape(1, -1), ws1, ws2)


def _topk_body(n_nodes, k, pre_ref, ch_ref, x_out_ref, idx_out_ref):
    npad = pre_ref.shape[1]
    f = pre_ref[...]
    iota = lax.broadcasted_iota(jnp.int32, (1, npad), 1).astype(jnp.float32)
    f = jnp.where(iota < n_nodes, jax.nn.sigmoid(f), -jnp.inf)

    def body(i, fcur):
        m = jnp.max(fcur)
        idxf = jnp.min(jnp.where(fcur == m, iota, jnp.float32(npad)))
        idx = idxf.astype(jnp.int32)
        rowv = ch_ref[pl.ds(idx, 1), :]
        x_out_ref[pl.ds(i, 1), :] = rowv * m
        idx_out_ref[pl.ds(i, 1), :] = jnp.zeros((1, _LANE), jnp.int32) + idx
        return jnp.where(iota == idxf, -jnp.inf, fcur)

    lax.fori_loop(0, k, body, f)


def _topk(pre, cluster_h, k):
    n = cluster_h.shape[0]
    npad = _rup(n, _LANE)
    pre_p = jnp.zeros((1, npad), jnp.float32).at[0, :n].set(pre)
    x_out, idx_out = pl.pallas_call(
        functools.partial(_topk_body, n, k),
        out_shape=(
            jax.ShapeDtypeStruct((k, cluster_h.shape[1]), jnp.float32),
            jax.ShapeDtypeStruct((k, _LANE), jnp.int32),
        ),
    )(pre_p, cluster_h)
    return x_out, idx_out[:, 0]


def kernel(x, edge_index, edge_weight, node_graph_index, Wg, bg, Wq, bq, Ws, bs,
           Wls, bls, Wlas, blas, Wlan):
    num_nodes, d = x.shape
    u = Wg.shape[1]
    k = _K

    loop = jnp.arange(num_nodes, dtype=edge_index.dtype)
    row = jnp.concatenate([edge_index[0], loop])
    col = jnp.concatenate([edge_index[1], loop])
    w = jnp.concatenate([edge_weight, jnp.ones((num_nodes,), x.dtype)])

    deg = jax.ops.segment_sum(w, row, num_segments=num_nodes)
    dinv = jnp.where(deg > 0, lax.rsqrt(deg), 0.0)
    nw = dinv[row] * w * dinv[col]
    hg = _mm(x, Wg)
    attention_h = jax.ops.segment_sum(nw[:, None] * hg[col], row,
                                      num_segments=num_nodes) + bg

    pooled = jax.ops.segment_max(attention_h[col], row, num_segments=num_nodes)
    qsns = _qsns(pooled, attention_h, Wq, bq, Ws[:u], Ws[u:])
    score = qsns[row, 0] + qsns[col, 1] + bs[0]
    score = jax.nn.leaky_relu(score, negative_slope=0.2)

    smax = jax.ops.segment_max(score, row, num_segments=num_nodes)
    sexp = jnp.exp(score - smax[row])
    ssum = jax.ops.segment_sum(sexp, row, num_segments=num_nodes)
    alpha = sexp / ssum[row]

    cluster_h = jax.ops.segment_sum(alpha[:, None] * attention_h[col], row,
                                    num_segments=num_nodes)

    wcat = jnp.concatenate([Wls, Wlas, Wlan], axis=1)
    bcat = jnp.concatenate([bls, blas, jnp.zeros((1,), jnp.float32)])
    lin = _mm(cluster_h, wcat, bcat)
    aggr = deg * lin[:, 1] - jax.ops.segment_sum(w * lin[:, 2][col], row,
                                                 num_segments=num_nodes)
    pre = lin[:, 0] + aggr

    topk_x, topk_idx = _topk(pre, cluster_h, k)
    topk_graph = node_graph_index[topk_idx]

    rev = jnp.full((num_nodes,), -1, dtype=jnp.int32)
    rev = rev.at[topk_idx].set(jnp.arange(k, dtype=jnp.int32))
    arow = rev[row]
    valid = arow >= 0
    arow_c = jnp.where(valid, arow, 0)
    aval = jnp.where(valid, alpha, 0.0)
    S = jnp.zeros((k, num_nodes), x.dtype).at[arow_c, col].add(aval)
    T = jax.ops.segment_sum(w[:, None] * S.T[col], row, num_segments=num_nodes)

    npad = _rup(num_nodes, _LANE)
    S_p = jnp.zeros((k, npad), jnp.float32).at[:, :num_nodes].set(S)
    T_p = jnp.zeros((npad, k), jnp.float32).at[:num_nodes, :].set(T)
    pooled_adj = _mm(S_p, T_p)

    return topk_x, pooled_adj, topk_graph
